# Initial kernel scaffold; baseline (speedup 1.0000x reference)
import jax
import jax.numpy as jnp
from jax.experimental import pallas as pl


def kernel(x, W, b):
    raise NotImplementedError("write your pallas kernel here")


# speedup vs baseline: 1.5195x; 1.5195x over previous
import functools

import jax
import jax.numpy as jnp
from jax.experimental import pallas as pl

_N = 8192
_IN = 256
_CHUNKS = 8
_CHUNK = _N // _CHUNKS
_K = 10


def _weldon_kernel(x_ref, w_ref, b_ref, o_ref):
    w = w_ref[...]
    rows = []
    for c in range(_CHUNKS):
        xc = x_ref[0, pl.ds(c * _CHUNK, _CHUNK), :]
        rows.append(
            jax.lax.dot_general(
                w, xc, (((1,), (1,)), ((), ())),
                preferred_element_type=jnp.float32,
            )
        )
    s = jnp.concatenate(rows, axis=0)

    iota = jax.lax.broadcasted_iota(jnp.int32, (_CHUNKS, _CHUNK), 0) * _CHUNK \
        + jax.lax.broadcasted_iota(jnp.int32, (_CHUNKS, _CHUNK), 1)
    big_i = jnp.int32(_N)
    ninf = jnp.float32(float("-inf"))
    pinf = jnp.float32(float("inf"))

    def extract(vals, kbest, largest):
        acc = jnp.zeros((1, 1), jnp.float32)
        cur = vals
        for _ in range(kbest):
            if largest:
                m = jnp.max(cur, axis=(0, 1), keepdims=True)
            else:
                m = jnp.min(cur, axis=(0, 1), keepdims=True)
            acc = acc + m
            hit = jnp.where(cur == m, iota, big_i)
            j = jnp.min(hit, axis=(0, 1), keepdims=True)
            cur = jnp.where(iota == j, ninf if largest else pinf, cur)
        return acc

    top_sum = extract(s, _K, True)
    bot_sum = extract(s, _K, False)
    mean = (top_sum + bot_sum) / jnp.float32(2 * _K) + b_ref[...]
    o_ref[0] = jax.nn.sigmoid(mean)


@jax.jit
def kernel(x, W, b):
    B = x.shape[0]
    out = pl.pallas_call(
        _weldon_kernel,
        grid=(B,),
        in_specs=[
            pl.BlockSpec((1, _N, _IN), lambda i: (i, 0, 0)),
            pl.BlockSpec((1, _IN), lambda i: (0, 0)),
            pl.BlockSpec((1, 1), lambda i: (0, 0)),
        ],
        out_specs=pl.BlockSpec((1, 1, 1), lambda i: (i, 0, 0)),
        out_shape=jax.ShapeDtypeStruct((B, 1, 1), jnp.float32),
    )(x, W, jnp.reshape(b, (1, 1)))
    return out.reshape(-1)


# speedup vs baseline: 2.6709x; 1.7577x over previous
import functools

import jax
import jax.numpy as jnp
from jax.experimental import pallas as pl

_N = 8192
_IN = 256
_CHUNKS = 8
_CHUNK = _N // _CHUNKS
_K = 10


def _weldon_kernel(x_ref, w_ref, b_ref, o_ref):
    w = w_ref[...]
    rows = []
    for c in range(_CHUNKS):
        xc = x_ref[0, pl.ds(c * _CHUNK, _CHUNK), :]
        rows.append(
            jax.lax.dot_general(
                w, xc, (((1,), (1,)), ((), ())),
                preferred_element_type=jnp.float32,
            )
        )
    s = jnp.concatenate(rows, axis=0)

    ninf = jnp.float32(float("-inf"))
    pinf = jnp.float32(float("inf"))

    def extract(vals, kbest, largest):
        acc = jnp.zeros((1, 1), jnp.float32)
        need = jnp.full((1, 1), kbest, jnp.float32)
        cur = vals
        for _ in range(kbest):
            if largest:
                m = jnp.max(cur, axis=(0, 1), keepdims=True)
            else:
                m = jnp.min(cur, axis=(0, 1), keepdims=True)
            eq = cur == m
            cnt = jnp.sum(jnp.where(eq, 1.0, 0.0).astype(jnp.float32),
                          axis=(0, 1), keepdims=True)
            take = jnp.minimum(cnt, need)
            acc = acc + jnp.where(take > 0, m * take, 0.0)
            need = need - take
            cur = jnp.where(eq, ninf if largest else pinf, cur)
        return acc

    top_sum = extract(s, _K, True)
    bot_sum = extract(s, _K, False)
    mean = (top_sum + bot_sum) / jnp.float32(2 * _K) + b_ref[...]
    o_ref[0] = jax.nn.sigmoid(mean)


@jax.jit
def kernel(x, W, b):
    B = x.shape[0]
    out = pl.pallas_call(
        _weldon_kernel,
        grid=(B,),
        in_specs=[
            pl.BlockSpec((1, _N, _IN), lambda i: (i, 0, 0)),
            pl.BlockSpec((1, _IN), lambda i: (0, 0)),
            pl.BlockSpec((1, 1), lambda i: (0, 0)),
        ],
        out_specs=pl.BlockSpec((1, 1, 1), lambda i: (i, 0, 0)),
        out_shape=jax.ShapeDtypeStruct((B, 1, 1), jnp.float32),
    )(x, W, jnp.reshape(b, (1, 1)))
    return out.reshape(-1)


# speedup vs baseline: 3.3200x; 1.2430x over previous
import functools

import jax
import jax.numpy as jnp
from jax.experimental import pallas as pl

_N = 8192
_IN = 256
_CHUNKS = 8
_CHUNK = _N // _CHUNKS
_K = 10


def _weldon_kernel(x_ref, w_ref, b_ref, o_ref):
    w = w_ref[...]
    rows = []
    for c in range(_CHUNKS):
        xc = x_ref[0, pl.ds(c * _CHUNK, _CHUNK), :]
        rows.append(
            jax.lax.dot_general(
                w, xc, (((1,), (1,)), ((), ())),
                preferred_element_type=jnp.float32,
            )
        )
    s = jnp.concatenate(rows, axis=0)

    ninf = jnp.float32(float("-inf"))
    pinf = jnp.float32(float("inf"))

    t_acc = jnp.zeros((1, 1), jnp.float32)
    b_acc = jnp.zeros((1, 1), jnp.float32)
    t_need = jnp.full((1, 1), _K, jnp.float32)
    b_need = jnp.full((1, 1), _K, jnp.float32)
    t_cur = s
    b_cur = s
    for _ in range(_K):
        tm = jnp.max(t_cur, axis=(0, 1), keepdims=True)
        bm = jnp.min(b_cur, axis=(0, 1), keepdims=True)
        t_eq = t_cur == tm
        b_eq = b_cur == bm
        t_cnt = jnp.sum(jnp.where(t_eq, 1.0, 0.0), axis=(0, 1), keepdims=True)
        b_cnt = jnp.sum(jnp.where(b_eq, 1.0, 0.0), axis=(0, 1), keepdims=True)
        t_take = jnp.minimum(t_cnt, t_need)
        b_take = jnp.minimum(b_cnt, b_need)
        t_acc = t_acc + jnp.where(t_take > 0, tm * t_take, 0.0)
        b_acc = b_acc + jnp.where(b_take > 0, bm * b_take, 0.0)
        t_need = t_need - t_take
        b_need = b_need - b_take
        t_cur = jnp.where(t_eq, ninf, t_cur)
        b_cur = jnp.where(b_eq, pinf, b_cur)

    mean = (t_acc + b_acc) / jnp.float32(2 * _K) + b_ref[...]
    o_ref[0] = jax.nn.sigmoid(mean)


@jax.jit
def kernel(x, W, b):
    B = x.shape[0]
    out = pl.pallas_call(
        _weldon_kernel,
        grid=(B,),
        in_specs=[
            pl.BlockSpec((1, _N, _IN), lambda i: (i, 0, 0)),
            pl.BlockSpec((1, _IN), lambda i: (0, 0)),
            pl.BlockSpec((1, 1), lambda i: (0, 0)),
        ],
        out_specs=pl.BlockSpec((1, 1, 1), lambda i: (i, 0, 0)),
        out_shape=jax.ShapeDtypeStruct((B, 1, 1), jnp.float32),
    )(x, W, jnp.reshape(b, (1, 1)))
    return out.reshape(-1)
